# baseline (speedup 1.0000x reference)
import jax
import jax.numpy as jnp
from jax import lax
from jax.experimental import pallas as pl
from jax.experimental.pallas import tpu as pltpu
from jax.experimental.pallas import tpu_sc as plsc

NUM_CORES = 2
NUM_SUBCORES = 16
NUM_WORKERS = NUM_CORES * NUM_SUBCORES
LANES = 16


def _glove_body(c0_hbm, c1_hbm, w0_hbm, w1_hbm, b0_hbm, b1_hbm, out_hbm,
                idx0_v, idx1_v, e0_v, e1_v, bb0_v, bb1_v, out_v, sem):
    b_per_w = idx0_v.shape[0]
    dim = e0_v.shape[1]
    nchunk = dim // LANES
    wid = lax.axis_index("s") * NUM_CORES + lax.axis_index("c")
    base = wid * b_per_w

    pltpu.sync_copy(c0_hbm.at[pl.ds(base, b_per_w)], idx0_v)
    pltpu.sync_copy(c1_hbm.at[pl.ds(base, b_per_w)], idx1_v)

    copies = [
        pltpu.async_copy(w0_hbm.at[idx0_v], e0_v, sem),
        pltpu.async_copy(w1_hbm.at[idx1_v], e1_v, sem),
        pltpu.async_copy(b0_hbm.at[idx0_v], bb0_v, sem),
        pltpu.async_copy(b1_hbm.at[idx1_v], bb1_v, sem),
    ]
    for cp in copies:
        cp.wait()

    def grp_body(g, carry):
        s = g * LANES
        rows = s + lax.iota(jnp.int32, LANES)
        acc = bb0_v[pl.ds(s, LANES)] + bb1_v[pl.ds(s, LANES)]
        for d in range(dim):
            cols = jnp.full((LANES,), d, jnp.int32)
            acc = acc + plsc.load_gather(e0_v, [rows, cols]) * plsc.load_gather(
                e1_v, [rows, cols])
        out_v[pl.ds(s, LANES)] = acc
        return carry

    lax.fori_loop(0, b_per_w // LANES, grp_body, 0)

    pltpu.sync_copy(out_v, out_hbm.at[pl.ds(base, b_per_w)])


def kernel(x, W0, W1, b0, b1):
    batch = x.shape[0]
    dim = W0.shape[1]
    b_per_w = batch // NUM_WORKERS
    codes0 = x[:, 0].astype(jnp.int32)
    codes1 = x[:, 1].astype(jnp.int32)
    b0v = b0.reshape(-1)
    b1v = b1.reshape(-1)

    mesh = plsc.VectorSubcoreMesh(core_axis_name="c", subcore_axis_name="s")
    run = pl.kernel(
        _glove_body,
        out_type=jax.ShapeDtypeStruct((batch,), jnp.float32),
        mesh=mesh,
        compiler_params=pltpu.CompilerParams(
            needs_layout_passes=False, use_tc_tiling_on_sc=False),
        scratch_types=[
            pltpu.VMEM((b_per_w,), jnp.int32),
            pltpu.VMEM((b_per_w,), jnp.int32),
            pltpu.VMEM((b_per_w, dim), jnp.float32),
            pltpu.VMEM((b_per_w, dim), jnp.float32),
            pltpu.VMEM((b_per_w,), jnp.float32),
            pltpu.VMEM((b_per_w,), jnp.float32),
            pltpu.VMEM((b_per_w,), jnp.float32),
            pltpu.SemaphoreType.DMA,
        ],
    )
    return run(codes0, codes1, W0, W1, b0v, b1v)
